# Initial kernel scaffold; baseline (speedup 1.0000x reference)
import jax
import jax.numpy as jnp
from jax.experimental import pallas as pl


def kernel(h_P, edge_index_pap, edge_index_psp, W_pap, al_pap, ar_pap, b_pap, W_psp, al_psp, ar_psp, b_psp, W1, b1, W2):
    raise NotImplementedError("write your pallas kernel here")


# speedup vs baseline: 1.0239x; 1.0239x over previous
import functools

import jax
import jax.numpy as jnp
from jax.experimental import pallas as pl
from jax.experimental.pallas import tpu as pltpu

N = 10000
IN_FEATS = 256
OUT_FEATS = 64
HEADS = 8
HID = HEADS * OUT_FEATS
SEM_HID = 128
E = 160000

_NB = 1000


def _proj_body(h_ref, Wp_ref, Alp_ref, Arp_ref, Ws_ref, Als_ref, Ars_ref,
               fp_ref, elp_ref, erp_ref, fs_ref, els_ref, ers_ref):
    h = h_ref[...]
    fp = jnp.dot(h, Wp_ref[...], preferred_element_type=jnp.float32)
    fs = jnp.dot(h, Ws_ref[...], preferred_element_type=jnp.float32)
    fp_ref[...] = fp
    fs_ref[...] = fs
    elp_ref[...] = jnp.dot(fp, Alp_ref[...], preferred_element_type=jnp.float32)
    erp_ref[...] = jnp.dot(fp, Arp_ref[...], preferred_element_type=jnp.float32)
    els_ref[...] = jnp.dot(fs, Als_ref[...], preferred_element_type=jnp.float32)
    ers_ref[...] = jnp.dot(fs, Ars_ref[...], preferred_element_type=jnp.float32)


def _projections(h_P, W_pap, Al_pap, Ar_pap, W_psp, Al_psp, Ar_psp):
    grid = (N // _NB,)
    bspec_h = pl.BlockSpec((_NB, IN_FEATS), lambda i: (i, 0))
    bspec_W = pl.BlockSpec((IN_FEATS, HID), lambda i: (0, 0))
    bspec_A = pl.BlockSpec((HID, HEADS), lambda i: (0, 0))
    bspec_f = pl.BlockSpec((_NB, HID), lambda i: (i, 0))
    bspec_e = pl.BlockSpec((_NB, HEADS), lambda i: (i, 0))
    out_shape = (
        jax.ShapeDtypeStruct((N, HID), jnp.float32),
        jax.ShapeDtypeStruct((N, HEADS), jnp.float32),
        jax.ShapeDtypeStruct((N, HEADS), jnp.float32),
        jax.ShapeDtypeStruct((N, HID), jnp.float32),
        jax.ShapeDtypeStruct((N, HEADS), jnp.float32),
        jax.ShapeDtypeStruct((N, HEADS), jnp.float32),
    )
    return pl.pallas_call(
        _proj_body,
        grid=grid,
        in_specs=[bspec_h, bspec_W, bspec_A, bspec_A, bspec_W, bspec_A, bspec_A],
        out_specs=(bspec_f, bspec_e, bspec_e, bspec_f, bspec_e, bspec_e),
        out_shape=out_shape,
    )(h_P, W_pap, Al_pap, Ar_pap, W_psp, Al_psp, Ar_psp)


def _edge_phase(feat, el, er, src, dst):
    x = el[src] + er[dst]
    e = jnp.maximum(x, 0.2 * x)
    ex = jnp.exp(e)
    s = jax.ops.segment_sum(ex, dst, num_segments=N)
    alpha = ex / (s[dst] + 1e-9)
    feath = feat.reshape(N, HEADS, OUT_FEATS)
    msg = feath[src] * alpha[:, :, None]
    out = jax.ops.segment_sum(msg, dst, num_segments=N)
    return out.reshape(N, HID)


def kernel(h_P, edge_index_pap, edge_index_psp, W_pap, al_pap, ar_pap, b_pap,
           W_psp, al_psp, ar_psp, b_psp, W1, b1, W2):
    eye = jnp.eye(HEADS, dtype=jnp.float32)
    Al_pap = (al_pap[:, :, None] * eye[:, None, :]).reshape(HID, HEADS)
    Ar_pap = (ar_pap[:, :, None] * eye[:, None, :]).reshape(HID, HEADS)
    Al_psp = (al_psp[:, :, None] * eye[:, None, :]).reshape(HID, HEADS)
    Ar_psp = (ar_psp[:, :, None] * eye[:, None, :]).reshape(HID, HEADS)

    fp, elp, erp, fs, els, ers = _projections(
        h_P, W_pap, Al_pap, Ar_pap, W_psp, Al_psp, Ar_psp)

    sp = edge_index_pap[0].astype(jnp.int32)
    dp = edge_index_pap[1].astype(jnp.int32)
    ss = edge_index_psp[0].astype(jnp.int32)
    ds = edge_index_psp[1].astype(jnp.int32)

    agg_pap = _edge_phase(fp, elp, erp, sp, dp)
    agg_psp = _edge_phase(fs, els, ers, ss, ds)

    emb_pap = jax.nn.elu(agg_pap + b_pap[None, :])
    emb_psp = jax.nn.elu(agg_psp + b_psp[None, :])

    stacked = jnp.stack([emb_pap, emb_psp], axis=1)
    w = (jnp.tanh(stacked @ W1 + b1) @ W2).mean(0)
    beta = jax.nn.softmax(w, axis=0)
    fused = (beta[None, :, :] * stacked).sum(1)
    return emb_pap, emb_psp, fused, beta


# speedup vs baseline: 14.7858x; 14.4414x over previous
import functools

import jax
import jax.numpy as jnp
from jax import lax
from jax.experimental import pallas as pl
from jax.experimental.pallas import tpu as pltpu
from jax.experimental.pallas import tpu_sc as plsc

N = 10000
IN_FEATS = 256
OUT_FEATS = 64
HEADS = 8
HID = HEADS * OUT_FEATS
SEM_HID = 128
E = 160000

_NB = 1000
_K = 128
_NSUB = 16
_NCORE = 2
_NBLK = E // _K
_BLK_LO = _NBLK // _NSUB
_NREM = _NBLK - _BLK_LO * _NSUB
_ZR = 1000


def _proj_body(h_ref, Wp_ref, Alp_ref, Arp_ref, Ws_ref, Als_ref, Ars_ref,
               fp_ref, elp_ref, erp_ref, fs_ref, els_ref, ers_ref):
    h = h_ref[...]
    fp = jnp.dot(h, Wp_ref[...], preferred_element_type=jnp.float32)
    fs = jnp.dot(h, Ws_ref[...], preferred_element_type=jnp.float32)
    fp_ref[...] = fp
    fs_ref[...] = fs
    elp_ref[...] = jnp.dot(fp, Alp_ref[...], preferred_element_type=jnp.float32)
    erp_ref[...] = jnp.dot(fp, Arp_ref[...], preferred_element_type=jnp.float32)
    els_ref[...] = jnp.dot(fs, Als_ref[...], preferred_element_type=jnp.float32)
    ers_ref[...] = jnp.dot(fs, Ars_ref[...], preferred_element_type=jnp.float32)


def _projections(h_P, W_pap, Al_pap, Ar_pap, W_psp, Al_psp, Ar_psp):
    grid = (N // _NB,)
    bspec_h = pl.BlockSpec((_NB, IN_FEATS), lambda i: (i, 0))
    bspec_W = pl.BlockSpec((IN_FEATS, HID), lambda i: (0, 0))
    bspec_A = pl.BlockSpec((HID, HEADS), lambda i: (0, 0))
    bspec_f = pl.BlockSpec((_NB, HID), lambda i: (i, 0))
    bspec_e = pl.BlockSpec((_NB, HEADS), lambda i: (i, 0))
    out_shape = (
        jax.ShapeDtypeStruct((N, HID), jnp.float32),
        jax.ShapeDtypeStruct((N, HEADS), jnp.float32),
        jax.ShapeDtypeStruct((N, HEADS), jnp.float32),
        jax.ShapeDtypeStruct((N, HID), jnp.float32),
        jax.ShapeDtypeStruct((N, HEADS), jnp.float32),
        jax.ShapeDtypeStruct((N, HEADS), jnp.float32),
    )
    return pl.pallas_call(
        _proj_body,
        grid=grid,
        in_specs=[bspec_h, bspec_W, bspec_A, bspec_A, bspec_W, bspec_A, bspec_A],
        out_specs=(bspec_f, bspec_e, bspec_e, bspec_f, bspec_e, bspec_e),
        out_shape=out_shape,
    )(h_P, W_pap, Al_pap, Ar_pap, W_psp, Al_psp, Ar_psp)


_GDN = lax.GatherDimensionNumbers(
    offset_dims=(), collapsed_slice_dims=(0,), start_index_map=(0,))


def _sc_edge_body(el2_hbm, er2_hbm, feat_hbm, src_hbm, dst_hbm, z128_hbm,
                  z16_hbm, Ut_hbm, s_hbm,
                  src_v, dst_v, a_v, b_v, ex_v, feat_v, U_sh, s_sh, sem):
    core = lax.axis_index("c")
    sub = lax.axis_index("s")
    nblk = jnp.where(sub < _NREM, _BLK_LO + 1, _BLK_LO)
    blk0 = sub * _BLK_LO + jnp.minimum(sub, _NREM)

    for p in range(2):
        g = 2 * core + p
        h0 = 2 * g
        hsel0 = jnp.full((16,), h0, jnp.int32)[:, None]
        hsel1 = jnp.full((16,), h0 + 1, jnp.int32)[:, None]

        @pl.when(sub < 10)
        def _zero_u():
            pltpu.sync_copy(z128_hbm, U_sh.at[pl.ds(sub * _ZR, _ZR)])
        if p == 0:
            @pl.when(sub < 10)
            def _zero_s():
                pltpu.sync_copy(z16_hbm, s_sh.at[pl.ds(sub * _ZR, _ZR)])
        plsc.subcore_barrier()

        def _block(j, carry):
            base = (blk0 + j) * _K
            pltpu.sync_copy(src_hbm.at[pl.ds(base, _K)], src_v)
            pltpu.sync_copy(dst_hbm.at[pl.ds(base, _K)], dst_v)
            pltpu.async_copy(el2_hbm.at[src_v], a_v, sem).wait()
            pltpu.async_copy(er2_hbm.at[dst_v], b_v, sem).wait()

            def _shift(i, c):
                src_v[pl.ds(i * 16, 16)] = src_v[pl.ds(i * 16, 16)] + g * N
                return c
            lax.fori_loop(0, _K // 16, _shift, 0)
            pltpu.async_copy(feat_hbm.at[src_v], feat_v, sem).wait()

            def _edge(k, c):
                x = a_v[k, pl.ds(0, 16)] + b_v[k, pl.ds(0, 16)]
                ex = jnp.exp(jnp.maximum(x, 0.2 * x))
                ex_v[k, pl.ds(0, 16)] = ex
                b0 = lax.gather(ex, hsel0, _GDN, (1,),
                                mode=lax.GatherScatterMode.PROMISE_IN_BOUNDS)
                b1 = lax.gather(ex, hsel1, _GDN, (1,),
                                mode=lax.GatherScatterMode.PROMISE_IN_BOUNDS)
                for v in range(8):
                    bv = b0 if v < 4 else b1
                    feat_v[k, pl.ds(v * 16, 16)] = (
                        feat_v[k, pl.ds(v * 16, 16)] * bv)
                return c
            lax.fori_loop(0, _K, _edge, 0)

            if p == 0:
                pltpu.sync_copy(ex_v, s_sh.at[dst_v], add=True)
            pltpu.sync_copy(feat_v, U_sh.at[dst_v], add=True)
            return carry
        lax.fori_loop(0, nblk, _block, 0)
        plsc.subcore_barrier()

        @pl.when(sub < 10)
        def _out_u():
            pltpu.sync_copy(U_sh.at[pl.ds(sub * _ZR, _ZR)],
                            Ut_hbm.at[pl.ds(g * N + sub * _ZR, _ZR)])
        if p == 0:
            @pl.when(jnp.logical_and(core == 0, sub < 10))
            def _out_s():
                pltpu.sync_copy(s_sh.at[pl.ds(sub * _ZR, _ZR)],
                                s_hbm.at[pl.ds(sub * _ZR, _ZR)])
        plsc.subcore_barrier()


_sc_edge = functools.partial(
    pl.kernel,
    _sc_edge_body,
    out_type=(
        jax.ShapeDtypeStruct((4 * N, 128), jnp.float32),
        jax.ShapeDtypeStruct((N, 16), jnp.float32),
    ),
    mesh=plsc.VectorSubcoreMesh(core_axis_name="c", subcore_axis_name="s"),
    scratch_types=[
        pltpu.VMEM((_K,), jnp.int32),
        pltpu.VMEM((_K,), jnp.int32),
        pltpu.VMEM((_K, 16), jnp.float32),
        pltpu.VMEM((_K, 16), jnp.float32),
        pltpu.VMEM((_K, 16), jnp.float32),
        pltpu.VMEM((_K, 128), jnp.float32),
        pltpu.VMEM_SHARED((N, 128), jnp.float32),
        pltpu.VMEM_SHARED((N, 16), jnp.float32),
        pltpu.SemaphoreType.DMA,
    ],
    compiler_params=pltpu.CompilerParams(use_tc_tiling_on_sc=False),
)()


def _edge_phase(feat, el, er, src, dst, z128, z16):
    feat_t = feat.reshape(N, 4, 128).transpose(1, 0, 2).reshape(4 * N, 128)
    el2 = jnp.tile(el, (1, 2))
    er2 = jnp.tile(er, (1, 2))
    Ut, s16 = _sc_edge(el2, er2, feat_t, src, dst, z128, z16)
    U = Ut.reshape(4, N, 128).transpose(1, 0, 2).reshape(N, HEADS, OUT_FEATS)
    return (U / (s16[:, :HEADS, None] + 1e-9)).reshape(N, HID)


def kernel(h_P, edge_index_pap, edge_index_psp, W_pap, al_pap, ar_pap, b_pap,
           W_psp, al_psp, ar_psp, b_psp, W1, b1, W2):
    eye = jnp.eye(HEADS, dtype=jnp.float32)
    Al_pap = (al_pap[:, :, None] * eye[:, None, :]).reshape(HID, HEADS)
    Ar_pap = (ar_pap[:, :, None] * eye[:, None, :]).reshape(HID, HEADS)
    Al_psp = (al_psp[:, :, None] * eye[:, None, :]).reshape(HID, HEADS)
    Ar_psp = (ar_psp[:, :, None] * eye[:, None, :]).reshape(HID, HEADS)

    fp, elp, erp, fs, els, ers = _projections(
        h_P, W_pap, Al_pap, Ar_pap, W_psp, Al_psp, Ar_psp)

    sp = edge_index_pap[0].astype(jnp.int32)
    dp = edge_index_pap[1].astype(jnp.int32)
    ss = edge_index_psp[0].astype(jnp.int32)
    ds = edge_index_psp[1].astype(jnp.int32)

    z128 = jnp.zeros((_ZR, 128), jnp.float32)
    z16 = jnp.zeros((_ZR, 16), jnp.float32)

    agg_pap = _edge_phase(fp, elp, erp, sp, dp, z128, z16)
    agg_psp = _edge_phase(fs, els, ers, ss, ds, z128, z16)

    emb_pap = jax.nn.elu(agg_pap + b_pap[None, :])
    emb_psp = jax.nn.elu(agg_psp + b_psp[None, :])

    stacked = jnp.stack([emb_pap, emb_psp], axis=1)
    w = (jnp.tanh(stacked @ W1 + b1) @ W2).mean(0)
    beta = jax.nn.softmax(w, axis=0)
    fused = (beta[None, :, :] * stacked).sum(1)
    return emb_pap, emb_psp, fused, beta


# speedup vs baseline: 17.5777x; 1.1888x over previous
import functools

import jax
import jax.numpy as jnp
from jax import lax
from jax.experimental import pallas as pl
from jax.experimental.pallas import tpu as pltpu
from jax.experimental.pallas import tpu_sc as plsc

N = 10000
IN_FEATS = 256
OUT_FEATS = 64
HEADS = 8
HID = HEADS * OUT_FEATS
SEM_HID = 128
E = 160000

_NB = 1000
_K = 128
_NSUB = 16
_NCORE = 2
_NBLK = E // _K
_BLK_LO = _NBLK // _NSUB
_NREM = _NBLK - _BLK_LO * _NSUB
_ZR = 1000


def _proj_body(h_ref, Wp_ref, Alp_ref, Arp_ref, Ws_ref, Als_ref, Ars_ref,
               fp_ref, elp_ref, erp_ref, fs_ref, els_ref, ers_ref):
    h = h_ref[...]
    fp = jnp.dot(h, Wp_ref[...], preferred_element_type=jnp.float32)
    fs = jnp.dot(h, Ws_ref[...], preferred_element_type=jnp.float32)
    fp_ref[...] = fp
    fs_ref[...] = fs
    elp_ref[...] = jnp.dot(fp, Alp_ref[...], preferred_element_type=jnp.float32)
    erp_ref[...] = jnp.dot(fp, Arp_ref[...], preferred_element_type=jnp.float32)
    els_ref[...] = jnp.dot(fs, Als_ref[...], preferred_element_type=jnp.float32)
    ers_ref[...] = jnp.dot(fs, Ars_ref[...], preferred_element_type=jnp.float32)


def _projections(h_P, W_pap, Al_pap, Ar_pap, W_psp, Al_psp, Ar_psp):
    grid = (N // _NB,)
    bspec_h = pl.BlockSpec((_NB, IN_FEATS), lambda i: (i, 0))
    bspec_W = pl.BlockSpec((IN_FEATS, HID), lambda i: (0, 0))
    bspec_A = pl.BlockSpec((HID, HEADS), lambda i: (0, 0))
    bspec_f = pl.BlockSpec((_NB, HID), lambda i: (i, 0))
    bspec_e = pl.BlockSpec((_NB, HEADS), lambda i: (i, 0))
    out_shape = (
        jax.ShapeDtypeStruct((N, HID), jnp.float32),
        jax.ShapeDtypeStruct((N, HEADS), jnp.float32),
        jax.ShapeDtypeStruct((N, HEADS), jnp.float32),
        jax.ShapeDtypeStruct((N, HID), jnp.float32),
        jax.ShapeDtypeStruct((N, HEADS), jnp.float32),
        jax.ShapeDtypeStruct((N, HEADS), jnp.float32),
    )
    return pl.pallas_call(
        _proj_body,
        grid=grid,
        in_specs=[bspec_h, bspec_W, bspec_A, bspec_A, bspec_W, bspec_A, bspec_A],
        out_specs=(bspec_f, bspec_e, bspec_e, bspec_f, bspec_e, bspec_e),
        out_shape=out_shape,
    )(h_P, W_pap, Al_pap, Ar_pap, W_psp, Al_psp, Ar_psp)


_GDN = lax.GatherDimensionNumbers(
    offset_dims=(), collapsed_slice_dims=(0,), start_index_map=(0,))


def _sc_edge_body(el2_hbm, er2_hbm, feat_hbm, src_hbm, dst_hbm, z128_hbm,
                  z16_hbm, Ut_hbm, s_hbm,
                  src_v, src2_v, dst_v, a_v, b_v, ex_v, feat_v, U_sh, s_sh,
                  sem_a, sem_b, sem_f):
    core = lax.axis_index("c")
    sub = lax.axis_index("s")
    nblk = jnp.where(sub < _NREM, _BLK_LO + 1, _BLK_LO)
    blk0 = sub * _BLK_LO + jnp.minimum(sub, _NREM)

    for p in range(2):
        g = 2 * core + p
        h0 = 2 * g
        hsel0 = jnp.full((16,), h0, jnp.int32)[:, None]
        hsel1 = jnp.full((16,), h0 + 1, jnp.int32)[:, None]

        @pl.when(sub < 10)
        def _zero_u():
            pltpu.sync_copy(z128_hbm, U_sh.at[pl.ds(sub * _ZR, _ZR)])
        if p == 0:
            @pl.when(sub < 10)
            def _zero_s():
                pltpu.sync_copy(z16_hbm, s_sh.at[pl.ds(sub * _ZR, _ZR)])
        plsc.subcore_barrier()

        def _block(j, carry):
            base = (blk0 + j) * _K
            pltpu.sync_copy(src_hbm.at[pl.ds(base, _K)], src_v)
            pltpu.sync_copy(dst_hbm.at[pl.ds(base, _K)], dst_v)
            cp_a = pltpu.async_copy(el2_hbm.at[src_v], a_v, sem_a)
            cp_b = pltpu.async_copy(er2_hbm.at[dst_v], b_v, sem_b)

            def _shift(i, c):
                src2_v[pl.ds(i * 16, 16)] = src_v[pl.ds(i * 16, 16)] + g * N
                return c
            lax.fori_loop(0, _K // 16, _shift, 0)
            cp_f = pltpu.async_copy(feat_hbm.at[src2_v], feat_v, sem_f)
            cp_a.wait()
            cp_b.wait()
            cp_f.wait()

            def _edge(k, c):
                x = a_v[k, pl.ds(0, 16)] + b_v[k, pl.ds(0, 16)]
                ex = jnp.exp(jnp.maximum(x, 0.2 * x))
                ex_v[k, pl.ds(0, 16)] = ex
                b0 = lax.gather(ex, hsel0, _GDN, (1,),
                                mode=lax.GatherScatterMode.PROMISE_IN_BOUNDS)
                b1 = lax.gather(ex, hsel1, _GDN, (1,),
                                mode=lax.GatherScatterMode.PROMISE_IN_BOUNDS)
                for v in range(8):
                    bv = b0 if v < 4 else b1
                    feat_v[k, pl.ds(v * 16, 16)] = (
                        feat_v[k, pl.ds(v * 16, 16)] * bv)
                return c
            lax.fori_loop(0, _K, _edge, 0)

            if p == 0:
                pltpu.sync_copy(ex_v, s_sh.at[dst_v], add=True)
            pltpu.sync_copy(feat_v, U_sh.at[dst_v], add=True)
            return carry
        lax.fori_loop(0, nblk, _block, 0)
        plsc.subcore_barrier()

        @pl.when(sub < 10)
        def _out_u():
            pltpu.sync_copy(U_sh.at[pl.ds(sub * _ZR, _ZR)],
                            Ut_hbm.at[pl.ds(g * N + sub * _ZR, _ZR)])
        if p == 0:
            @pl.when(jnp.logical_and(core == 0, sub < 10))
            def _out_s():
                pltpu.sync_copy(s_sh.at[pl.ds(sub * _ZR, _ZR)],
                                s_hbm.at[pl.ds(sub * _ZR, _ZR)])
        plsc.subcore_barrier()


_sc_edge = functools.partial(
    pl.kernel,
    _sc_edge_body,
    out_type=(
        jax.ShapeDtypeStruct((4 * N, 128), jnp.float32),
        jax.ShapeDtypeStruct((N, 16), jnp.float32),
    ),
    mesh=plsc.VectorSubcoreMesh(core_axis_name="c", subcore_axis_name="s"),
    scratch_types=[
        pltpu.VMEM((_K,), jnp.int32),
        pltpu.VMEM((_K,), jnp.int32),
        pltpu.VMEM((_K,), jnp.int32),
        pltpu.VMEM((_K, 16), jnp.float32),
        pltpu.VMEM((_K, 16), jnp.float32),
        pltpu.VMEM((_K, 16), jnp.float32),
        pltpu.VMEM((_K, 128), jnp.float32),
        pltpu.VMEM_SHARED((N, 128), jnp.float32),
        pltpu.VMEM_SHARED((N, 16), jnp.float32),
        pltpu.SemaphoreType.DMA,
        pltpu.SemaphoreType.DMA,
        pltpu.SemaphoreType.DMA,
    ],
    compiler_params=pltpu.CompilerParams(use_tc_tiling_on_sc=False),
)()


def _edge_phase(feat, el, er, src, dst, z128, z16):
    feat_t = feat.reshape(N, 4, 128).transpose(1, 0, 2).reshape(4 * N, 128)
    el2 = jnp.tile(el, (1, 2))
    er2 = jnp.tile(er, (1, 2))
    Ut, s16 = _sc_edge(el2, er2, feat_t, src, dst, z128, z16)
    U = Ut.reshape(4, N, 128).transpose(1, 0, 2).reshape(N, HEADS, OUT_FEATS)
    return (U / (s16[:, :HEADS, None] + 1e-9)).reshape(N, HID)


def kernel(h_P, edge_index_pap, edge_index_psp, W_pap, al_pap, ar_pap, b_pap,
           W_psp, al_psp, ar_psp, b_psp, W1, b1, W2):
    eye = jnp.eye(HEADS, dtype=jnp.float32)
    Al_pap = (al_pap[:, :, None] * eye[:, None, :]).reshape(HID, HEADS)
    Ar_pap = (ar_pap[:, :, None] * eye[:, None, :]).reshape(HID, HEADS)
    Al_psp = (al_psp[:, :, None] * eye[:, None, :]).reshape(HID, HEADS)
    Ar_psp = (ar_psp[:, :, None] * eye[:, None, :]).reshape(HID, HEADS)

    fp, elp, erp, fs, els, ers = _projections(
        h_P, W_pap, Al_pap, Ar_pap, W_psp, Al_psp, Ar_psp)

    sp = edge_index_pap[0].astype(jnp.int32)
    dp = edge_index_pap[1].astype(jnp.int32)
    ss = edge_index_psp[0].astype(jnp.int32)
    ds = edge_index_psp[1].astype(jnp.int32)

    z128 = jnp.zeros((_ZR, 128), jnp.float32)
    z16 = jnp.zeros((_ZR, 16), jnp.float32)

    agg_pap = _edge_phase(fp, elp, erp, sp, dp, z128, z16)
    agg_psp = _edge_phase(fs, els, ers, ss, ds, z128, z16)

    emb_pap = jax.nn.elu(agg_pap + b_pap[None, :])
    emb_psp = jax.nn.elu(agg_psp + b_psp[None, :])

    stacked = jnp.stack([emb_pap, emb_psp], axis=1)
    w = (jnp.tanh(stacked @ W1 + b1) @ W2).mean(0)
    beta = jax.nn.softmax(w, axis=0)
    fused = (beta[None, :, :] * stacked).sum(1)
    return emb_pap, emb_psp, fused, beta
